# baseline (speedup 1.0000x reference)
import functools

import jax
import jax.numpy as jnp
from jax import lax
from jax.experimental import pallas as pl
from jax.experimental.pallas import tpu as pltpu
from jax.experimental.pallas import tpu_sc as plsc

FULL_DIM = 2048
N_ROWS = 8192
L = 16
CS = 32
N_CHUNKS = FULL_DIM // CS
NTR = N_ROWS * N_CHUNKS
NC = 2
NS = 16
NW = NC * NS
XROWS_PER_W = N_ROWS // NW
RB = 8
TRB = RB * N_CHUNKS
N_BLKS = XROWS_PER_W // RB
IDX_MINOR = 128
G_DMAS = TRB // IDX_MINOR
IDX_MAJOR = XROWS_PER_W * N_CHUNKS // IDX_MINOR


def _make_permute_kernel():
    mesh = plsc.VectorSubcoreMesh(core_axis_name="c", subcore_axis_name="s")

    @functools.partial(
        pl.kernel,
        mesh=mesh,
        out_type=jax.ShapeDtypeStruct((NTR, CS), jnp.float32),
        compiler_params=pltpu.CompilerParams(
            needs_layout_passes=False, use_tc_tiling_on_sc=False),
        scratch_types=[
            pltpu.VMEM((FULL_DIM,), jnp.int32),
            pltpu.VMEM((N_CHUNKS,), jnp.int32),
            pltpu.VMEM((IDX_MAJOR, IDX_MINOR), jnp.int32),
            pltpu.VMEM((TRB, CS), jnp.float32),
            pltpu.VMEM((TRB, CS), jnp.float32),
            pltpu.SemaphoreType.DMA,
            pltpu.SemaphoreType.DMA,
            pltpu.SemaphoreType.DMA,
            pltpu.SemaphoreType.DMA,
        ],
    )
    def permute_rows(x_hbm, perm_hbm, out_hbm, perm_v, inv_v, sidx,
                     buf_a, buf_b, isem_a, isem_b, osem_a, osem_b):
        wid = lax.axis_index("s") * NC + lax.axis_index("c")
        xr0 = wid * XROWS_PER_W
        tr0 = xr0 * N_CHUNKS

        pltpu.sync_copy(perm_hbm, perm_v)

        iota = lax.iota(jnp.int32, L)
        for k in range(4):
            csrc_k = plsc.load_gather(perm_v, [(iota + L * k) * CS]) >> 5
            plsc.store_scatter(inv_v, [csrc_k], iota + L * k)

        inv = [inv_v[pl.ds(L * k, L)] for k in range(4)]

        def fill(t, carry):
            for h in range(2):
                base = (xr0 + 2 * t + h) * N_CHUNKS
                for k in range(4):
                    sidx[t, pl.ds(h * 64 + k * L, L)] = inv[k] + base
            return carry

        lax.fori_loop(0, XROWS_PER_W // 2, fill, 0)

        bufs = (buf_a, buf_b)
        isems = (isem_a, isem_b)
        osems = (osem_a, osem_b)

        def fire_in(b):
            p = b % 2
            return pltpu.async_copy(
                x_hbm.at[pl.ds(tr0 + b * TRB, TRB)], bufs[p], isems[p])

        def fire_scatters(b):
            p = b % 2
            return [
                pltpu.async_copy(
                    bufs[p].at[pl.ds(a * IDX_MINOR, IDX_MINOR)],
                    out_hbm.at[sidx.at[b * G_DMAS + a]],
                    osems[p])
                for a in range(G_DMAS)
            ]

        scatters = [None, None]
        pending_in = fire_in(0)
        for b in range(N_BLKS):
            p = b % 2
            pending_in.wait()
            next_in = None
            if b + 1 < N_BLKS:
                if scatters[(b + 1) % 2] is not None:
                    for cp in scatters[(b + 1) % 2]:
                        cp.wait()
                next_in = fire_in(b + 1)
            scatters[p] = fire_scatters(b)
            pending_in = next_in
        for group in scatters:
            for cp in group:
                cp.wait()

    return permute_rows


_PERMUTE_ROWS = _make_permute_kernel()


def kernel(x, permute):
    table = jnp.reshape(x, (NTR, CS))
    out = _PERMUTE_ROWS(table, permute)
    return jnp.reshape(out, (N_ROWS, FULL_DIM))


# speedup vs baseline: 1.2181x; 1.2181x over previous
import functools

import jax
import jax.numpy as jnp
from jax import lax
from jax.experimental import pallas as pl
from jax.experimental.pallas import tpu as pltpu
from jax.experimental.pallas import tpu_sc as plsc

FULL_DIM = 2048
N_ROWS = 8192
L = 16
NC = 2
NS = 16
NW = NC * NS
XROWS_PER_W = N_ROWS // NW
RB = 8
N_BLKS = XROWS_PER_W // RB
N_PAIRS = N_BLKS // 2
GROUPS = FULL_DIM // L
MC = 4
MPC = GROUPS // MC


def _make_permute_kernel():
    mesh = plsc.VectorSubcoreMesh(core_axis_name="c", subcore_axis_name="s")

    @functools.partial(
        pl.kernel,
        mesh=mesh,
        out_type=jax.ShapeDtypeStruct((N_ROWS, FULL_DIM), jnp.float32),
        compiler_params=pltpu.CompilerParams(needs_layout_passes=False),
        scratch_types=[
            pltpu.VMEM((FULL_DIM,), jnp.int32),
            pltpu.VMEM((RB, FULL_DIM), jnp.float32),
            pltpu.VMEM((RB, FULL_DIM), jnp.float32),
            pltpu.VMEM((RB, FULL_DIM), jnp.float32),
            pltpu.VMEM((RB, FULL_DIM), jnp.float32),
            pltpu.SemaphoreType.DMA,
            pltpu.SemaphoreType.DMA,
            pltpu.SemaphoreType.DMA,
            pltpu.SemaphoreType.DMA,
        ],
    )
    def permute_rows(x_hbm, perm_hbm, out_hbm, perm_v,
                     in_a, in_b, out_a, out_b,
                     isem_a, isem_b, osem_a, osem_b):
        wid = lax.axis_index("s") * NC + lax.axis_index("c")
        row0 = wid * XROWS_PER_W

        pltpu.sync_copy(perm_hbm, perm_v)

        def permute_block(src, dst):
            for mc in range(MC):
                pvecs = [perm_v[pl.ds((mc * MPC + m) * L, L)]
                         for m in range(MPC)]

                def row_body(r, carry):
                    rvec = jnp.full((L,), 0, jnp.int32) + r
                    for m in range(MPC):
                        dst[r, pl.ds((mc * MPC + m) * L, L)] = (
                            plsc.load_gather(src, [rvec, pvecs[m]])
                        )
                    return carry

                lax.fori_loop(0, RB, row_body, 0)

        def pair_body(i, carry):
            r_a = row0 + (2 * i) * RB
            r_b = r_a + RB
            pltpu.async_copy(x_hbm.at[pl.ds(r_b, RB)], in_b, isem_b)
            pltpu.make_async_copy(x_hbm.at[pl.ds(r_a, RB)], in_a, isem_a).wait()

            @pl.when(i > 0)
            def _():
                pltpu.make_async_copy(
                    out_a, out_hbm.at[pl.ds(r_a, RB)], osem_a).wait()

            permute_block(in_a, out_a)
            pltpu.async_copy(out_a, out_hbm.at[pl.ds(r_a, RB)], osem_a)

            @pl.when(i < N_PAIRS - 1)
            def _():
                pltpu.async_copy(
                    x_hbm.at[pl.ds(r_b + RB, RB)], in_a, isem_a)

            pltpu.make_async_copy(x_hbm.at[pl.ds(r_b, RB)], in_b, isem_b).wait()

            @pl.when(i > 0)
            def _():
                pltpu.make_async_copy(
                    out_b, out_hbm.at[pl.ds(r_b, RB)], osem_b).wait()

            permute_block(in_b, out_b)
            pltpu.async_copy(out_b, out_hbm.at[pl.ds(r_b, RB)], osem_b)
            return carry

        pltpu.async_copy(x_hbm.at[pl.ds(row0, RB)], in_a, isem_a)
        lax.fori_loop(0, N_PAIRS, pair_body, 0)
        pltpu.make_async_copy(out_a, out_hbm.at[pl.ds(row0, RB)], osem_a).wait()
        pltpu.make_async_copy(out_b, out_hbm.at[pl.ds(row0, RB)], osem_b).wait()

    return permute_rows


_PERMUTE_ROWS = _make_permute_kernel()


def kernel(x, permute):
    return _PERMUTE_ROWS(x, permute)
